# baseline (speedup 1.0000x reference)
import dataclasses
import functools

import jax
import jax.numpy as jnp
from jax import lax
from jax.experimental import pallas as pl
from jax.experimental.pallas import tpu as pltpu
from jax.experimental.pallas import tpu_sc as plsc

_N = 10000
_E = 160000
_DLAT = 128
_DIM = 3
_L = 2
_EMB = 8
_NTYPES = 20
_MAXR = 50.0

_NC = 2
_NS = 16
_NW = _NC * _NS
_CH = 128
_NSUB = 40
_EPW = _CH * _NSUB
_EP = _EPW * _NW
_NP = 10240
_RPS = _NP // _NS
_LANES = 16
_CK = 4
_NSUBC = _NSUB // _CK
_EPC = _EP // _CK
_EPWC = _EPW // _CK


def _sc_typegather(attr, idx_s, idx_d):
    mesh = plsc.VectorSubcoreMesh(core_axis_name="c", subcore_axis_name="s")
    out = [jax.ShapeDtypeStruct((_EP,), jnp.int32),
           jax.ShapeDtypeStruct((_EP,), jnp.int32)]
    cp = pltpu.CompilerParams()
    if "needs_layout_passes" in pltpu.CompilerParams.__dataclass_fields__:
        cp = dataclasses.replace(cp, needs_layout_passes=False)

    @functools.partial(
        pl.kernel, out_type=out, mesh=mesh, compiler_params=cp,
        scratch_types=[pltpu.VMEM((_NP,), jnp.int32),
                       pltpu.VMEM((_NSUB, _CH), jnp.int32),
                       pltpu.VMEM((_NSUB, _CH), jnp.int32),
                       pltpu.VMEM((_EPW,), jnp.int32),
                       pltpu.VMEM((_EPW,), jnp.int32)],
    )
    def k(a_hbm, is_hbm, id_hbm, os_hbm, od_hbm,
          a_v, iv_s, iv_d, ov_s, ov_d):
        wid = lax.axis_index("s") * _NC + lax.axis_index("c")
        base = wid * _EPW
        pltpu.sync_copy(a_hbm, a_v)
        pltpu.sync_copy(is_hbm.at[wid], iv_s)
        pltpu.sync_copy(id_hbm.at[wid], iv_d)

        @pl.loop(0, _NSUB)
        def _(j):
            @pl.loop(0, _CH, step=_LANES)
            def _(c):
                ts = plsc.load_gather(a_v, [iv_s[j, pl.ds(c, _LANES)]])
                td = plsc.load_gather(a_v, [iv_d[j, pl.ds(c, _LANES)]])
                ov_s[pl.ds(j * _CH + c, _LANES)] = ts
                ov_d[pl.ds(j * _CH + c, _LANES)] = td

        pltpu.sync_copy(ov_s, os_hbm.at[pl.ds(base, _EPW)])
        pltpu.sync_copy(ov_d, od_hbm.at[pl.ds(base, _EPW)])

    return k(attr, idx_s, idx_d)


def _sc_gather(tbl, idx_s, idx_d):
    mesh = plsc.VectorSubcoreMesh(core_axis_name="c", subcore_axis_name="s")
    out = [jax.ShapeDtypeStruct((_EPC, _DLAT), jnp.float32),
           jax.ShapeDtypeStruct((_EPC, _DLAT), jnp.float32)]

    @functools.partial(
        pl.kernel, out_type=out, mesh=mesh,
        scratch_types=[pltpu.VMEM((_NSUBC, _CH), jnp.int32),
                       pltpu.VMEM((_NSUBC, _CH), jnp.int32),
                       pltpu.VMEM((2, _CH, _DLAT), jnp.float32),
                       pltpu.VMEM((2, _CH, _DLAT), jnp.float32),
                       pltpu.SemaphoreType.DMA,
                       pltpu.SemaphoreType.DMA,
                       pltpu.SemaphoreType.DMA,
                       pltpu.SemaphoreType.DMA],
    )
    def k(tbl_hbm, is_hbm, id_hbm, os_hbm, od_hbm,
          iv_s, iv_d, rows_s, rows_d, sem_s0, sem_s1, sem_d0, sem_d1):
        wid = lax.axis_index("s") * _NC + lax.axis_index("c")
        base = wid * _EPWC
        pltpu.sync_copy(is_hbm.at[wid], iv_s)
        pltpu.sync_copy(id_hbm.at[wid], iv_d)
        sems = ((sem_s0, sem_d0), (sem_s1, sem_d1))

        def issue(j, b):
            pltpu.make_async_copy(tbl_hbm.at[iv_s.at[j]], rows_s.at[b],
                                  sems[b][0]).start()
            pltpu.make_async_copy(tbl_hbm.at[iv_d.at[j]], rows_d.at[b],
                                  sems[b][1]).start()

        def drain(j, b):
            pltpu.make_async_copy(tbl_hbm.at[iv_s.at[j]], rows_s.at[b],
                                  sems[b][0]).wait()
            pltpu.make_async_copy(tbl_hbm.at[iv_d.at[j]], rows_d.at[b],
                                  sems[b][1]).wait()
            off = base + j * _CH
            pltpu.sync_copy(rows_s.at[b], os_hbm.at[pl.ds(off, _CH)])
            pltpu.sync_copy(rows_d.at[b], od_hbm.at[pl.ds(off, _CH)])

        issue(0, 0)

        @pl.loop(0, _NSUBC - 2, step=2)
        def _(j):
            issue(j + 1, 1)
            drain(j, 0)
            issue(j + 2, 0)
            drain(j + 1, 1)

        issue(_NSUBC - 1, 1)
        drain(_NSUBC - 2, 0)
        drain(_NSUBC - 1, 1)

    return k(tbl, idx_s, idx_d)


def _sc_scatter(qps, pqs, idx_d, idx_s, zeros):
    mesh = plsc.VectorSubcoreMesh(core_axis_name="c", subcore_axis_name="s")
    out = jax.ShapeDtypeStruct((_NC, _NP, _DLAT), jnp.float32)

    @functools.partial(
        pl.kernel, out_type=out, mesh=mesh,
        scratch_types=[pltpu.VMEM((_NSUB, _CH), jnp.int32),
                       pltpu.VMEM((_NSUB, _CH), jnp.int32),
                       pltpu.VMEM((_CH, _DLAT), jnp.float32),
                       pltpu.VMEM((_CH, _DLAT), jnp.float32),
                       pltpu.VMEM_SHARED((_NP, _DLAT), jnp.float32)],
    )
    def k(qp0, qp1, qp2, qp3, pq0, pq1, pq2, pq3, id_hbm, is_hbm, z_hbm,
          o_hbm, iv_d, iv_s, rows_q, rows_p, acc):
        cid = lax.axis_index("c")
        sid = lax.axis_index("s")
        wid = sid * _NC + cid
        base = wid * _EPWC
        rbase = sid * _RPS
        pltpu.sync_copy(z_hbm.at[cid, pl.ds(rbase, _RPS)],
                        acc.at[pl.ds(rbase, _RPS)])
        pltpu.sync_copy(id_hbm.at[wid], iv_d)
        pltpu.sync_copy(is_hbm.at[wid], iv_s)
        plsc.subcore_barrier()

        for c, (qp_hbm, pq_hbm) in enumerate(
                zip((qp0, qp1, qp2, qp3), (pq0, pq1, pq2, pq3))):
            @pl.loop(0, _NSUBC)
            def _(j, qp_hbm=qp_hbm, pq_hbm=pq_hbm, c=c):
                off = base + j * _CH
                pltpu.sync_copy(qp_hbm.at[pl.ds(off, _CH)], rows_q)
                pltpu.sync_copy(pq_hbm.at[pl.ds(off, _CH)], rows_p)
                pltpu.sync_copy(rows_q, acc.at[iv_d.at[c * _NSUBC + j]],
                                add=True)
                pltpu.sync_copy(rows_p, acc.at[iv_s.at[c * _NSUBC + j]],
                                add=True)

        plsc.subcore_barrier()
        pltpu.sync_copy(acc.at[pl.ds(rbase, _RPS)],
                        o_hbm.at[cid, pl.ds(rbase, _RPS)])

    return k(*qps, *pqs, idx_d, idx_s, zeros)


_BE = 1024


def _onehot(types):
    return (types == lax.broadcasted_iota(jnp.int32, (1, _NTYPES), 1)
            ).astype(jnp.float32)


def _dense_body(ys_ref, yd_ref, tps_ref, tpd_ref, e_ref, ws_ref, wd_ref,
                wc_ref, b_ref, w_ref, k_ref, base_ref, qp_ref, pq_ref):
    ys = ys_ref[...]
    yd = yd_ref[...]
    dif = ys - yd
    dv = jnp.dot(dif, k_ref[...], preferred_element_type=jnp.float32)
    r = jnp.sqrt(jnp.sum(dv * dv, axis=1, keepdims=True))
    u = 2.0 * (r / _MAXR - 1.0)
    c = (1.0 - jnp.cos(jnp.pi * u)) * 0.5
    c = jnp.where(u > 0.0, 0.0, c)
    c = jnp.where(u < -1.0, 1.0, c)
    w = c / r
    emb = e_ref[...]
    ps = jnp.dot(emb, ws_ref[...], preferred_element_type=jnp.float32)
    pd = jnp.dot(emb, wd_ref[...], preferred_element_type=jnp.float32)
    pre = (jnp.dot(_onehot(tps_ref[...]), ps,
                   preferred_element_type=jnp.float32)
           + jnp.dot(_onehot(tpd_ref[...]), pd,
                     preferred_element_type=jnp.float32)
           + w * wc_ref[...] + b_ref[...])
    wg = pre * jax.nn.sigmoid(pre)
    gx = wg * dif
    ax = 0.5 * wg * (ys + yd)
    dxe = jnp.concatenate([gx, ax, gx * ax, gx * gx, ax * ax], axis=1)
    wm = w_ref[...]
    t = jnp.dot(jnp.tanh(dxe), wm, preferred_element_type=jnp.float32)
    t = t - jnp.mean(t, axis=1, keepdims=True)
    t = t / jnp.sqrt(jnp.sum(t * t, axis=1, keepdims=True) + 1e-3)
    t = jnp.dot(jnp.tanh(t), wm, preferred_element_type=jnp.float32)
    d2 = jnp.tanh(t)
    q = wg * d2[:, :_DLAT]
    p = 0.5 * wg * (d2[:, _DLAT:2 * _DLAT] + d2[:, 2 * _DLAT:3 * _DLAT]
                    + d2[:, 3 * _DLAT:4 * _DLAT] + d2[:, 4 * _DLAT:])
    eidx = (lax.broadcasted_iota(jnp.int32, (_BE, 1), 0)
            + pl.program_id(0) * _BE + base_ref[0, 0])
    mask = eidx < _E
    qp_ref[...] = jnp.where(mask, q + p, 0.0)
    pq_ref[...] = jnp.where(mask, p - q, 0.0)


def _tc_dense(ys, yd, tps, tpd, emb_table, ws, wd, wcol, bias, dlwt, pu_k,
              base):
    return pl.pallas_call(
        _dense_body,
        grid=(_EPC // _BE,),
        in_specs=[
            pl.BlockSpec((_BE, _DLAT), lambda i: (i, 0)),
            pl.BlockSpec((_BE, _DLAT), lambda i: (i, 0)),
            pl.BlockSpec((_BE, 1), lambda i: (i, 0)),
            pl.BlockSpec((_BE, 1), lambda i: (i, 0)),
            pl.BlockSpec((_NTYPES, _EMB), lambda i: (0, 0)),
            pl.BlockSpec((_EMB, _DLAT), lambda i: (0, 0)),
            pl.BlockSpec((_EMB, _DLAT), lambda i: (0, 0)),
            pl.BlockSpec((1, _DLAT), lambda i: (0, 0)),
            pl.BlockSpec((1, _DLAT), lambda i: (0, 0)),
            pl.BlockSpec((5 * _DLAT, 5 * _DLAT), lambda i: (0, 0)),
            pl.BlockSpec((_DLAT, _DIM), lambda i: (0, 0)),
            pl.BlockSpec((1, 1), lambda i: (0, 0)),
        ],
        out_specs=[pl.BlockSpec((_BE, _DLAT), lambda i: (i, 0)),
                   pl.BlockSpec((_BE, _DLAT), lambda i: (i, 0))],
        out_shape=[jax.ShapeDtypeStruct((_EPC, _DLAT), jnp.float32),
                   jax.ShapeDtypeStruct((_EPC, _DLAT), jnp.float32)],
    )(ys, yd, tps, tpd, emb_table, ws, wd, wcol, bias, dlwt, pu_k, base)


_BN = 1024


def _prep_body(x_ref, k_ref, y_ref):
    y_ref[...] = jnp.dot(x_ref[...], k_ref[...],
                         preferred_element_type=jnp.float32)


def _tc_prep(xp, pu_kt):
    return pl.pallas_call(
        _prep_body,
        grid=(_NP // _BN,),
        in_specs=[
            pl.BlockSpec((_BN, _DIM), lambda i: (i, 0)),
            pl.BlockSpec((_DIM, _DLAT), lambda i: (0, 0)),
        ],
        out_specs=pl.BlockSpec((_BN, _DLAT), lambda i: (i, 0)),
        out_shape=jax.ShapeDtypeStruct((_NP, _DLAT), jnp.float32),
    )(xp, pu_kt)


def _update_body(acc_ref, y_ref, yo_ref, h_ref, k_ref, yn_ref, x_ref):
    accsum = acc_ref[0] + acc_ref[1]
    yn = 2.0 * y_ref[...] - yo_ref[...] - h_ref[0, 0] * accsum
    yn_ref[...] = yn
    x_ref[...] = jnp.dot(yn, k_ref[...], preferred_element_type=jnp.float32)


def _tc_update(accs, y, y_old, hsq, pu_k):
    return pl.pallas_call(
        _update_body,
        grid=(_NP // _BN,),
        in_specs=[
            pl.BlockSpec((_NC, _BN, _DLAT), lambda i: (0, i, 0)),
            pl.BlockSpec((_BN, _DLAT), lambda i: (i, 0)),
            pl.BlockSpec((_BN, _DLAT), lambda i: (i, 0)),
            pl.BlockSpec((1, 1), lambda i: (0, 0)),
            pl.BlockSpec((_DLAT, _DIM), lambda i: (0, 0)),
        ],
        out_specs=[pl.BlockSpec((_BN, _DLAT), lambda i: (i, 0)),
                   pl.BlockSpec((_BN, _DIM), lambda i: (i, 0))],
        out_shape=[jax.ShapeDtypeStruct((_NP, _DLAT), jnp.float32),
                   jax.ShapeDtypeStruct((_NP, _DIM), jnp.float32)],
    )(accs, y, y_old, hsq, pu_k)


def kernel(x, batch, node_attr, edge_src, edge_dst, emb_table, fc1_W, fc1_b,
           dl_W, h, PU_K):
    del batch
    f32 = jnp.float32
    xp = jnp.pad(x.astype(f32), ((0, _NP - _N), (0, 0)))
    attr = jnp.pad(node_attr, (0, _NP - _N))
    srcf = jnp.pad(edge_src, (0, _EP - _E), constant_values=_NP - 1)
    dstf = jnp.pad(edge_dst, (0, _EP - _E), constant_values=_NP - 1)
    srcp = srcf.reshape(_NW, _NSUB, _CH)
    dstp = dstf.reshape(_NW, _NSUB, _CH)
    srcc = srcf.reshape(_CK, _NW, _NSUBC, _CH)
    dstc = dstf.reshape(_CK, _NW, _NSUBC, _CH)
    zeros = jnp.zeros((_NC, _NP, _DLAT), f32)
    pu_kt = PU_K.T

    tps, tpd = _sc_typegather(attr, srcp, dstp)
    tps = tps.reshape(_EP, 1)
    tpd = tpd.reshape(_EP, 1)
    y = _tc_prep(xp, pu_kt)
    y_old = y
    xn = None
    for i in range(_L):
        ws = fc1_W[i][:, :_EMB].T
        wd = fc1_W[i][:, _EMB:2 * _EMB].T
        wcol = fc1_W[i][:, 2 * _EMB].reshape(1, _DLAT)
        bias = fc1_b[i].reshape(1, _DLAT)
        dlwt = dl_W[i].T
        qps = []
        pqs = []
        for c in range(_CK):
            ys, yd = _sc_gather(y, srcc[c], dstc[c])
            base = jnp.full((1, 1), c * _EPC, jnp.int32)
            qp, pq = _tc_dense(ys, yd, tps[c * _EPC:(c + 1) * _EPC],
                               tpd[c * _EPC:(c + 1) * _EPC], emb_table,
                               ws, wd, wcol, bias, dlwt, PU_K, base)
            qps.append(qp)
            pqs.append(pq)
        part = _sc_scatter(qps, pqs, dstp, srcp, zeros)
        hsq = (h[i] * h[i]).reshape(1, 1)
        yn, xn = _tc_update(part, y, y_old, hsq, PU_K)
        y_old = y
        y = yn
    return xn[:_N]


# speedup vs baseline: 1.0554x; 1.0554x over previous
import dataclasses
import functools

import jax
import jax.numpy as jnp
from jax import lax
from jax.experimental import pallas as pl
from jax.experimental.pallas import tpu as pltpu
from jax.experimental.pallas import tpu_sc as plsc

_N = 10000
_E = 160000
_DLAT = 128
_DIM = 3
_L = 2
_EMB = 8
_NTYPES = 20
_MAXR = 50.0

_NC = 2
_NS = 16
_NW = _NC * _NS
_CH = 128
_NSUB = 40
_EPW = _CH * _NSUB
_EP = _EPW * _NW
_NP = 10240
_RPS = _NP // _NS
_LANES = 16
_CK = 4
_NSUBC = _NSUB // _CK
_EPC = _EP // _CK
_EPWC = _EPW // _CK


def _sc_typegather(attr, idx_s, idx_d):
    mesh = plsc.VectorSubcoreMesh(core_axis_name="c", subcore_axis_name="s")
    out = [jax.ShapeDtypeStruct((_EP,), jnp.int32),
           jax.ShapeDtypeStruct((_EP,), jnp.int32)]
    cp = pltpu.CompilerParams()
    if "needs_layout_passes" in pltpu.CompilerParams.__dataclass_fields__:
        cp = dataclasses.replace(cp, needs_layout_passes=False)

    @functools.partial(
        pl.kernel, out_type=out, mesh=mesh, compiler_params=cp,
        scratch_types=[pltpu.VMEM((_NP,), jnp.int32),
                       pltpu.VMEM((_NSUB, _CH), jnp.int32),
                       pltpu.VMEM((_NSUB, _CH), jnp.int32),
                       pltpu.VMEM((_EPW,), jnp.int32),
                       pltpu.VMEM((_EPW,), jnp.int32)],
    )
    def k(a_hbm, is_hbm, id_hbm, os_hbm, od_hbm,
          a_v, iv_s, iv_d, ov_s, ov_d):
        wid = lax.axis_index("s") * _NC + lax.axis_index("c")
        base = wid * _EPW
        pltpu.sync_copy(a_hbm, a_v)
        pltpu.sync_copy(is_hbm.at[wid], iv_s)
        pltpu.sync_copy(id_hbm.at[wid], iv_d)

        @pl.loop(0, _NSUB)
        def _(j):
            @pl.loop(0, _CH, step=_LANES)
            def _(c):
                ts = plsc.load_gather(a_v, [iv_s[j, pl.ds(c, _LANES)]])
                td = plsc.load_gather(a_v, [iv_d[j, pl.ds(c, _LANES)]])
                ov_s[pl.ds(j * _CH + c, _LANES)] = ts
                ov_d[pl.ds(j * _CH + c, _LANES)] = td

        pltpu.sync_copy(ov_s, os_hbm.at[pl.ds(base, _EPW)])
        pltpu.sync_copy(ov_d, od_hbm.at[pl.ds(base, _EPW)])

    return k(attr, idx_s, idx_d)


def _sc_gather(tbl, idx_s, idx_d):
    mesh = plsc.VectorSubcoreMesh(core_axis_name="c", subcore_axis_name="s")
    out = [jax.ShapeDtypeStruct((_EPC, _DLAT), jnp.float32),
           jax.ShapeDtypeStruct((_EPC, _DLAT), jnp.float32)]

    @functools.partial(
        pl.kernel, out_type=out, mesh=mesh,
        scratch_types=[pltpu.VMEM((_NSUBC, _CH), jnp.int32),
                       pltpu.VMEM((_NSUBC, _CH), jnp.int32),
                       pltpu.VMEM((2, _CH, _DLAT), jnp.float32),
                       pltpu.VMEM((2, _CH, _DLAT), jnp.float32),
                       pltpu.SemaphoreType.DMA,
                       pltpu.SemaphoreType.DMA,
                       pltpu.SemaphoreType.DMA,
                       pltpu.SemaphoreType.DMA],
    )
    def k(tbl_hbm, is_hbm, id_hbm, os_hbm, od_hbm,
          iv_s, iv_d, rows_s, rows_d, sem_s0, sem_s1, sem_d0, sem_d1):
        wid = lax.axis_index("s") * _NC + lax.axis_index("c")
        base = wid * _EPWC
        pltpu.sync_copy(is_hbm.at[wid], iv_s)
        pltpu.sync_copy(id_hbm.at[wid], iv_d)
        sems = ((sem_s0, sem_d0), (sem_s1, sem_d1))

        def issue(j, b):
            pltpu.make_async_copy(tbl_hbm.at[iv_s.at[j]], rows_s.at[b],
                                  sems[b][0]).start()
            pltpu.make_async_copy(tbl_hbm.at[iv_d.at[j]], rows_d.at[b],
                                  sems[b][1]).start()

        def drain(j, b):
            pltpu.make_async_copy(tbl_hbm.at[iv_s.at[j]], rows_s.at[b],
                                  sems[b][0]).wait()
            pltpu.make_async_copy(tbl_hbm.at[iv_d.at[j]], rows_d.at[b],
                                  sems[b][1]).wait()
            off = base + j * _CH
            pltpu.sync_copy(rows_s.at[b], os_hbm.at[pl.ds(off, _CH)])
            pltpu.sync_copy(rows_d.at[b], od_hbm.at[pl.ds(off, _CH)])

        issue(0, 0)

        @pl.loop(0, _NSUBC - 2, step=2)
        def _(j):
            issue(j + 1, 1)
            drain(j, 0)
            issue(j + 2, 0)
            drain(j + 1, 1)

        issue(_NSUBC - 1, 1)
        drain(_NSUBC - 2, 0)
        drain(_NSUBC - 1, 1)

    return k(tbl, idx_s, idx_d)


def _sc_scatter(qp, pq, idx_d, idx_s, prev):
    mesh = plsc.VectorSubcoreMesh(core_axis_name="c", subcore_axis_name="s")
    out = jax.ShapeDtypeStruct((_NC, _NP, _DLAT), jnp.float32)

    @functools.partial(
        pl.kernel, out_type=out, mesh=mesh,
        scratch_types=[pltpu.VMEM((_NSUBC, _CH), jnp.int32),
                       pltpu.VMEM((_NSUBC, _CH), jnp.int32),
                       pltpu.VMEM((_CH, _DLAT), jnp.float32),
                       pltpu.VMEM((_CH, _DLAT), jnp.float32),
                       pltpu.VMEM_SHARED((_NP, _DLAT), jnp.float32)],
    )
    def k(qp_hbm, pq_hbm, id_hbm, is_hbm, z_hbm, o_hbm,
          iv_d, iv_s, rows_q, rows_p, acc):
        cid = lax.axis_index("c")
        sid = lax.axis_index("s")
        wid = sid * _NC + cid
        base = wid * _EPWC
        rbase = sid * _RPS
        pltpu.sync_copy(z_hbm.at[cid, pl.ds(rbase, _RPS)],
                        acc.at[pl.ds(rbase, _RPS)])
        pltpu.sync_copy(id_hbm.at[wid], iv_d)
        pltpu.sync_copy(is_hbm.at[wid], iv_s)
        plsc.subcore_barrier()

        @pl.loop(0, _NSUBC)
        def _(j):
            off = base + j * _CH
            pltpu.sync_copy(qp_hbm.at[pl.ds(off, _CH)], rows_q)
            pltpu.sync_copy(pq_hbm.at[pl.ds(off, _CH)], rows_p)
            pltpu.sync_copy(rows_q, acc.at[iv_d.at[j]], add=True)
            pltpu.sync_copy(rows_p, acc.at[iv_s.at[j]], add=True)

        plsc.subcore_barrier()
        pltpu.sync_copy(acc.at[pl.ds(rbase, _RPS)],
                        o_hbm.at[cid, pl.ds(rbase, _RPS)])

    return k(qp, pq, idx_d, idx_s, prev)


_BE = 1024


def _onehot(types):
    return (types == lax.broadcasted_iota(jnp.int32, (1, _NTYPES), 1)
            ).astype(jnp.float32)


def _dense_body(ys_ref, yd_ref, tps_ref, tpd_ref, e_ref, ws_ref, wd_ref,
                wc_ref, b_ref, w_ref, k_ref, base_ref, qp_ref, pq_ref):
    ys = ys_ref[...]
    yd = yd_ref[...]
    dif = ys - yd
    dv = jnp.dot(dif, k_ref[...], preferred_element_type=jnp.float32)
    r = jnp.sqrt(jnp.sum(dv * dv, axis=1, keepdims=True))
    u = 2.0 * (r / _MAXR - 1.0)
    c = (1.0 - jnp.cos(jnp.pi * u)) * 0.5
    c = jnp.where(u > 0.0, 0.0, c)
    c = jnp.where(u < -1.0, 1.0, c)
    w = c / r
    emb = e_ref[...]
    ps = jnp.dot(emb, ws_ref[...], preferred_element_type=jnp.float32)
    pd = jnp.dot(emb, wd_ref[...], preferred_element_type=jnp.float32)
    pre = (jnp.dot(_onehot(tps_ref[...]), ps,
                   preferred_element_type=jnp.float32)
           + jnp.dot(_onehot(tpd_ref[...]), pd,
                     preferred_element_type=jnp.float32)
           + w * wc_ref[...] + b_ref[...])
    wg = pre * jax.nn.sigmoid(pre)
    bf = jnp.bfloat16
    wgb = wg.astype(bf)
    gx = wgb * dif.astype(bf)
    ax = 0.5 * wgb * (ys + yd).astype(bf)
    dxe = jnp.concatenate([gx, ax, gx * ax, gx * gx, ax * ax], axis=1)
    wm = w_ref[...]
    t = jnp.dot(jnp.tanh(dxe), wm, preferred_element_type=jnp.float32)
    t = t - jnp.mean(t, axis=1, keepdims=True)
    t = t * lax.rsqrt(jnp.sum(t * t, axis=1, keepdims=True) + 1e-3)
    t = jnp.dot(jnp.tanh(t.astype(bf)), wm, preferred_element_type=jnp.float32)
    d2 = jnp.tanh(t.astype(bf))
    q = (wgb * d2[:, :_DLAT]).astype(jnp.float32)
    p = (0.5 * wgb * (d2[:, _DLAT:2 * _DLAT] + d2[:, 2 * _DLAT:3 * _DLAT]
                      + d2[:, 3 * _DLAT:4 * _DLAT] + d2[:, 4 * _DLAT:])
         ).astype(jnp.float32)
    eidx = (lax.broadcasted_iota(jnp.int32, (_BE, 1), 0)
            + pl.program_id(0) * _BE + base_ref[0, 0])
    mask = eidx < _E
    qp_ref[...] = jnp.where(mask, q + p, 0.0)
    pq_ref[...] = jnp.where(mask, p - q, 0.0)


def _tc_dense(ys, yd, tps, tpd, emb_table, ws, wd, wcol, bias, dlwt, pu_k,
              base):
    return pl.pallas_call(
        _dense_body,
        grid=(_EPC // _BE,),
        in_specs=[
            pl.BlockSpec((_BE, _DLAT), lambda i: (i, 0)),
            pl.BlockSpec((_BE, _DLAT), lambda i: (i, 0)),
            pl.BlockSpec((_BE, 1), lambda i: (i, 0)),
            pl.BlockSpec((_BE, 1), lambda i: (i, 0)),
            pl.BlockSpec((_NTYPES, _EMB), lambda i: (0, 0)),
            pl.BlockSpec((_EMB, _DLAT), lambda i: (0, 0)),
            pl.BlockSpec((_EMB, _DLAT), lambda i: (0, 0)),
            pl.BlockSpec((1, _DLAT), lambda i: (0, 0)),
            pl.BlockSpec((1, _DLAT), lambda i: (0, 0)),
            pl.BlockSpec((5 * _DLAT, 5 * _DLAT), lambda i: (0, 0)),
            pl.BlockSpec((_DLAT, _DIM), lambda i: (0, 0)),
            pl.BlockSpec((1, 1), lambda i: (0, 0)),
        ],
        out_specs=[pl.BlockSpec((_BE, _DLAT), lambda i: (i, 0)),
                   pl.BlockSpec((_BE, _DLAT), lambda i: (i, 0))],
        out_shape=[jax.ShapeDtypeStruct((_EPC, _DLAT), jnp.float32),
                   jax.ShapeDtypeStruct((_EPC, _DLAT), jnp.float32)],
    )(ys, yd, tps, tpd, emb_table, ws, wd, wcol, bias,
      dlwt.astype(jnp.bfloat16), pu_k, base)


_BN = 1024


def _prep_body(x_ref, k_ref, y_ref):
    y_ref[...] = jnp.dot(x_ref[...], k_ref[...],
                         preferred_element_type=jnp.float32)


def _tc_prep(xp, pu_kt):
    return pl.pallas_call(
        _prep_body,
        grid=(_NP // _BN,),
        in_specs=[
            pl.BlockSpec((_BN, _DIM), lambda i: (i, 0)),
            pl.BlockSpec((_DIM, _DLAT), lambda i: (0, 0)),
        ],
        out_specs=pl.BlockSpec((_BN, _DLAT), lambda i: (i, 0)),
        out_shape=jax.ShapeDtypeStruct((_NP, _DLAT), jnp.float32),
    )(xp, pu_kt)


def _update_body(acc_ref, y_ref, yo_ref, h_ref, k_ref, yn_ref, x_ref):
    accsum = acc_ref[0] + acc_ref[1]
    yn = 2.0 * y_ref[...] - yo_ref[...] - h_ref[0, 0] * accsum
    yn_ref[...] = yn
    x_ref[...] = jnp.dot(yn, k_ref[...], preferred_element_type=jnp.float32)


def _tc_update(accs, y, y_old, hsq, pu_k):
    return pl.pallas_call(
        _update_body,
        grid=(_NP // _BN,),
        in_specs=[
            pl.BlockSpec((_NC, _BN, _DLAT), lambda i: (0, i, 0)),
            pl.BlockSpec((_BN, _DLAT), lambda i: (i, 0)),
            pl.BlockSpec((_BN, _DLAT), lambda i: (i, 0)),
            pl.BlockSpec((1, 1), lambda i: (0, 0)),
            pl.BlockSpec((_DLAT, _DIM), lambda i: (0, 0)),
        ],
        out_specs=[pl.BlockSpec((_BN, _DLAT), lambda i: (i, 0)),
                   pl.BlockSpec((_BN, _DIM), lambda i: (i, 0))],
        out_shape=[jax.ShapeDtypeStruct((_NP, _DLAT), jnp.float32),
                   jax.ShapeDtypeStruct((_NP, _DIM), jnp.float32)],
    )(accs, y, y_old, hsq, pu_k)


def kernel(x, batch, node_attr, edge_src, edge_dst, emb_table, fc1_W, fc1_b,
           dl_W, h, PU_K):
    del batch
    f32 = jnp.float32
    xp = jnp.pad(x.astype(f32), ((0, _NP - _N), (0, 0)))
    attr = jnp.pad(node_attr, (0, _NP - _N))
    srcf = jnp.pad(edge_src, (0, _EP - _E), constant_values=_NP - 1)
    dstf = jnp.pad(edge_dst, (0, _EP - _E), constant_values=_NP - 1)
    srcp = srcf.reshape(_NW, _NSUB, _CH)
    dstp = dstf.reshape(_NW, _NSUB, _CH)
    srcc = srcf.reshape(_CK, _NW, _NSUBC, _CH)
    dstc = dstf.reshape(_CK, _NW, _NSUBC, _CH)
    zeros = jnp.zeros((_NC, _NP, _DLAT), f32)
    pu_kt = PU_K.T

    tps, tpd = _sc_typegather(attr, srcp, dstp)
    tps = tps.reshape(_EP, 1)
    tpd = tpd.reshape(_EP, 1)
    y = _tc_prep(xp, pu_kt)
    y_old = y
    xn = None
    for i in range(_L):
        ws = fc1_W[i][:, :_EMB].T
        wd = fc1_W[i][:, _EMB:2 * _EMB].T
        wcol = fc1_W[i][:, 2 * _EMB].reshape(1, _DLAT)
        bias = fc1_b[i].reshape(1, _DLAT)
        dlwt = dl_W[i].T
        part = zeros
        for c in range(_CK):
            ys, yd = _sc_gather(y, srcc[c], dstc[c])
            base = jnp.full((1, 1), c * _EPC, jnp.int32)
            qp, pq = _tc_dense(ys, yd, tps[c * _EPC:(c + 1) * _EPC],
                               tpd[c * _EPC:(c + 1) * _EPC], emb_table,
                               ws, wd, wcol, bias, dlwt, PU_K, base)
            part = _sc_scatter(qp, pq, dstc[c], srcc[c], part)
        hsq = (h[i] * h[i]).reshape(1, 1)
        yn, xn = _tc_update(part, y, y_old, hsq, PU_K)
        y_old = y
        y = yn
    return xn[:_N]


# speedup vs baseline: 1.2031x; 1.1399x over previous
import dataclasses
import functools

import jax
import jax.numpy as jnp
from jax import lax
from jax.experimental import pallas as pl
from jax.experimental.pallas import tpu as pltpu
from jax.experimental.pallas import tpu_sc as plsc

_N = 10000
_E = 160000
_DLAT = 128
_DIM = 3
_L = 2
_EMB = 8
_NTYPES = 20
_MAXR = 50.0

_NC = 2
_NS = 16
_NW = _NC * _NS
_CH = 128
_NSUB = 40
_EPW = _CH * _NSUB
_EP = _EPW * _NW
_NP = 10240
_RPS = _NP // _NS
_LANES = 16
_CK = 4
_NSUBC = _NSUB // _CK
_EPC = _EP // _CK
_EPWC = _EPW // _CK


def _sc_typegather(attr, idx_s, idx_d):
    mesh = plsc.VectorSubcoreMesh(core_axis_name="c", subcore_axis_name="s")
    out = [jax.ShapeDtypeStruct((_EP,), jnp.int32),
           jax.ShapeDtypeStruct((_EP,), jnp.int32)]
    cp = pltpu.CompilerParams()
    if "needs_layout_passes" in pltpu.CompilerParams.__dataclass_fields__:
        cp = dataclasses.replace(cp, needs_layout_passes=False)

    @functools.partial(
        pl.kernel, out_type=out, mesh=mesh, compiler_params=cp,
        scratch_types=[pltpu.VMEM((_NP,), jnp.int32),
                       pltpu.VMEM((_NSUB, _CH), jnp.int32),
                       pltpu.VMEM((_NSUB, _CH), jnp.int32),
                       pltpu.VMEM((_EPW,), jnp.int32),
                       pltpu.VMEM((_EPW,), jnp.int32)],
    )
    def k(a_hbm, is_hbm, id_hbm, os_hbm, od_hbm,
          a_v, iv_s, iv_d, ov_s, ov_d):
        wid = lax.axis_index("s") * _NC + lax.axis_index("c")
        base = wid * _EPW
        pltpu.sync_copy(a_hbm, a_v)
        pltpu.sync_copy(is_hbm.at[wid], iv_s)
        pltpu.sync_copy(id_hbm.at[wid], iv_d)

        @pl.loop(0, _NSUB)
        def _(j):
            @pl.loop(0, _CH, step=_LANES)
            def _(c):
                ts = plsc.load_gather(a_v, [iv_s[j, pl.ds(c, _LANES)]])
                td = plsc.load_gather(a_v, [iv_d[j, pl.ds(c, _LANES)]])
                ov_s[pl.ds(j * _CH + c, _LANES)] = ts
                ov_d[pl.ds(j * _CH + c, _LANES)] = td

        pltpu.sync_copy(ov_s, os_hbm.at[pl.ds(base, _EPW)])
        pltpu.sync_copy(ov_d, od_hbm.at[pl.ds(base, _EPW)])

    return k(attr, idx_s, idx_d)


def _sc_gather(tbl, idx_s, idx_d):
    mesh = plsc.VectorSubcoreMesh(core_axis_name="c", subcore_axis_name="s")
    out = [jax.ShapeDtypeStruct((_EPC, _DLAT), jnp.float32),
           jax.ShapeDtypeStruct((_EPC, _DLAT), jnp.float32)]

    @functools.partial(
        pl.kernel, out_type=out, mesh=mesh,
        scratch_types=[pltpu.VMEM((_NSUBC, _CH), jnp.int32),
                       pltpu.VMEM((_NSUBC, _CH), jnp.int32),
                       pltpu.VMEM((2, _CH, _DLAT), jnp.float32),
                       pltpu.VMEM((2, _CH, _DLAT), jnp.float32),
                       pltpu.SemaphoreType.DMA,
                       pltpu.SemaphoreType.DMA,
                       pltpu.SemaphoreType.DMA,
                       pltpu.SemaphoreType.DMA],
    )
    def k(tbl_hbm, is_hbm, id_hbm, os_hbm, od_hbm,
          iv_s, iv_d, rows_s, rows_d, sem_s0, sem_s1, sem_d0, sem_d1):
        wid = lax.axis_index("s") * _NC + lax.axis_index("c")
        base = wid * _EPWC
        pltpu.sync_copy(is_hbm.at[wid], iv_s)
        pltpu.sync_copy(id_hbm.at[wid], iv_d)
        sems = ((sem_s0, sem_d0), (sem_s1, sem_d1))

        def issue(j, b):
            pltpu.make_async_copy(tbl_hbm.at[iv_s.at[j]], rows_s.at[b],
                                  sems[b][0]).start()
            pltpu.make_async_copy(tbl_hbm.at[iv_d.at[j]], rows_d.at[b],
                                  sems[b][1]).start()

        def drain(j, b):
            pltpu.make_async_copy(tbl_hbm.at[iv_s.at[j]], rows_s.at[b],
                                  sems[b][0]).wait()
            pltpu.make_async_copy(tbl_hbm.at[iv_d.at[j]], rows_d.at[b],
                                  sems[b][1]).wait()
            off = base + j * _CH
            pltpu.sync_copy(rows_s.at[b], os_hbm.at[pl.ds(off, _CH)])
            pltpu.sync_copy(rows_d.at[b], od_hbm.at[pl.ds(off, _CH)])

        issue(0, 0)

        @pl.loop(0, _NSUBC - 2, step=2)
        def _(j):
            issue(j + 1, 1)
            drain(j, 0)
            issue(j + 2, 0)
            drain(j + 1, 1)

        issue(_NSUBC - 1, 1)
        drain(_NSUBC - 2, 0)
        drain(_NSUBC - 1, 1)

    return k(tbl, idx_s, idx_d)


def _sc_scatter(qpa, qpb, pqa, pqb, idx_d, idx_s, prev):
    mesh = plsc.VectorSubcoreMesh(core_axis_name="c", subcore_axis_name="s")
    out = jax.ShapeDtypeStruct((_NC, _NP, _DLAT), jnp.float32)

    @functools.partial(
        pl.kernel, out_type=out, mesh=mesh,
        scratch_types=[pltpu.VMEM((2 * _NSUBC, _CH), jnp.int32),
                       pltpu.VMEM((2 * _NSUBC, _CH), jnp.int32),
                       pltpu.VMEM((_CH, _DLAT), jnp.float32),
                       pltpu.VMEM((_CH, _DLAT), jnp.float32),
                       pltpu.VMEM_SHARED((_NP, _DLAT), jnp.float32)],
    )
    def k(qpa_hbm, qpb_hbm, pqa_hbm, pqb_hbm, id_hbm, is_hbm, z_hbm, o_hbm,
          iv_d, iv_s, rows_q, rows_p, acc):
        cid = lax.axis_index("c")
        sid = lax.axis_index("s")
        wid = sid * _NC + cid
        base = wid * _EPWC
        rbase = sid * _RPS
        pltpu.sync_copy(z_hbm.at[cid, pl.ds(rbase, _RPS)],
                        acc.at[pl.ds(rbase, _RPS)])
        pltpu.sync_copy(id_hbm.at[wid], iv_d)
        pltpu.sync_copy(is_hbm.at[wid], iv_s)
        plsc.subcore_barrier()

        for half, (qp_hbm, pq_hbm) in enumerate(((qpa_hbm, pqa_hbm),
                                                 (qpb_hbm, pqb_hbm))):
            @pl.loop(0, _NSUBC)
            def _(j, qp_hbm=qp_hbm, pq_hbm=pq_hbm, half=half):
                off = base + j * _CH
                pltpu.sync_copy(qp_hbm.at[pl.ds(off, _CH)], rows_q)
                pltpu.sync_copy(pq_hbm.at[pl.ds(off, _CH)], rows_p)
                pltpu.sync_copy(rows_q,
                                acc.at[iv_d.at[half * _NSUBC + j]], add=True)
                pltpu.sync_copy(rows_p,
                                acc.at[iv_s.at[half * _NSUBC + j]], add=True)

        plsc.subcore_barrier()
        pltpu.sync_copy(acc.at[pl.ds(rbase, _RPS)],
                        o_hbm.at[cid, pl.ds(rbase, _RPS)])

    return k(qpa, qpb, pqa, pqb, idx_d, idx_s, prev)


_BE = 1024


def _onehot(types):
    return (types == lax.broadcasted_iota(jnp.int32, (1, _NTYPES), 1)
            ).astype(jnp.float32)


def _dense_body(ys_ref, yd_ref, tps_ref, tpd_ref, e_ref, ws_ref, wd_ref,
                wc_ref, b_ref, w_ref, k_ref, base_ref, qp_ref, pq_ref):
    ys = ys_ref[...]
    yd = yd_ref[...]
    dif = ys - yd
    dv = jnp.dot(dif, k_ref[...], preferred_element_type=jnp.float32)
    r = jnp.sqrt(jnp.sum(dv * dv, axis=1, keepdims=True))
    u = 2.0 * (r / _MAXR - 1.0)
    c = (1.0 - jnp.cos(jnp.pi * u)) * 0.5
    c = jnp.where(u > 0.0, 0.0, c)
    c = jnp.where(u < -1.0, 1.0, c)
    w = c / r
    emb = e_ref[...]
    ps = jnp.dot(emb, ws_ref[...], preferred_element_type=jnp.float32)
    pd = jnp.dot(emb, wd_ref[...], preferred_element_type=jnp.float32)
    pre = (jnp.dot(_onehot(tps_ref[...]), ps,
                   preferred_element_type=jnp.float32)
           + jnp.dot(_onehot(tpd_ref[...]), pd,
                     preferred_element_type=jnp.float32)
           + w * wc_ref[...] + b_ref[...])
    wg = pre * jax.nn.sigmoid(pre)
    bf = jnp.bfloat16
    wgb = wg.astype(bf)
    gx = wgb * dif.astype(bf)
    ax = 0.5 * wgb * (ys + yd).astype(bf)
    dxe = jnp.concatenate([gx, ax, gx * ax, gx * gx, ax * ax], axis=1)
    wm = w_ref[...]
    t = jnp.dot(jnp.tanh(dxe), wm, preferred_element_type=jnp.float32)
    t = t - jnp.mean(t, axis=1, keepdims=True)
    t = t * lax.rsqrt(jnp.sum(t * t, axis=1, keepdims=True) + 1e-3)
    t = jnp.dot(jnp.tanh(t.astype(bf)), wm, preferred_element_type=jnp.float32)
    d2 = jnp.tanh(t.astype(bf))
    q = (wgb * d2[:, :_DLAT]).astype(jnp.float32)
    p = (0.5 * wgb * (d2[:, _DLAT:2 * _DLAT] + d2[:, 2 * _DLAT:3 * _DLAT]
                      + d2[:, 3 * _DLAT:4 * _DLAT] + d2[:, 4 * _DLAT:])
         ).astype(jnp.float32)
    eidx = (lax.broadcasted_iota(jnp.int32, (_BE, 1), 0)
            + pl.program_id(0) * _BE + base_ref[0, 0])
    mask = eidx < _E
    qp_ref[...] = jnp.where(mask, q + p, 0.0)
    pq_ref[...] = jnp.where(mask, p - q, 0.0)


def _tc_dense(ys, yd, tps, tpd, emb_table, ws, wd, wcol, bias, dlwt, pu_k,
              base):
    return pl.pallas_call(
        _dense_body,
        grid=(_EPC // _BE,),
        in_specs=[
            pl.BlockSpec((_BE, _DLAT), lambda i: (i, 0)),
            pl.BlockSpec((_BE, _DLAT), lambda i: (i, 0)),
            pl.BlockSpec((_BE, 1), lambda i: (i, 0)),
            pl.BlockSpec((_BE, 1), lambda i: (i, 0)),
            pl.BlockSpec((_NTYPES, _EMB), lambda i: (0, 0)),
            pl.BlockSpec((_EMB, _DLAT), lambda i: (0, 0)),
            pl.BlockSpec((_EMB, _DLAT), lambda i: (0, 0)),
            pl.BlockSpec((1, _DLAT), lambda i: (0, 0)),
            pl.BlockSpec((1, _DLAT), lambda i: (0, 0)),
            pl.BlockSpec((5 * _DLAT, 5 * _DLAT), lambda i: (0, 0)),
            pl.BlockSpec((_DLAT, _DIM), lambda i: (0, 0)),
            pl.BlockSpec((1, 1), lambda i: (0, 0)),
        ],
        out_specs=[pl.BlockSpec((_BE, _DLAT), lambda i: (i, 0)),
                   pl.BlockSpec((_BE, _DLAT), lambda i: (i, 0))],
        out_shape=[jax.ShapeDtypeStruct((_EPC, _DLAT), jnp.float32),
                   jax.ShapeDtypeStruct((_EPC, _DLAT), jnp.float32)],
    )(ys, yd, tps, tpd, emb_table, ws, wd, wcol, bias,
      dlwt.astype(jnp.bfloat16), pu_k, base)


_BN = 1024


def _prep_body(x_ref, k_ref, y_ref):
    y_ref[...] = jnp.dot(x_ref[...], k_ref[...],
                         preferred_element_type=jnp.float32)


def _tc_prep(xp, pu_kt):
    return pl.pallas_call(
        _prep_body,
        grid=(_NP // _BN,),
        in_specs=[
            pl.BlockSpec((_BN, _DIM), lambda i: (i, 0)),
            pl.BlockSpec((_DIM, _DLAT), lambda i: (0, 0)),
        ],
        out_specs=pl.BlockSpec((_BN, _DLAT), lambda i: (i, 0)),
        out_shape=jax.ShapeDtypeStruct((_NP, _DLAT), jnp.float32),
    )(xp, pu_kt)


def _update_body(acc_ref, y_ref, yo_ref, h_ref, k_ref, yn_ref, x_ref):
    accsum = acc_ref[0] + acc_ref[1]
    yn = 2.0 * y_ref[...] - yo_ref[...] - h_ref[0, 0] * accsum
    yn_ref[...] = yn
    x_ref[...] = jnp.dot(yn, k_ref[...], preferred_element_type=jnp.float32)


def _tc_update(accs, y, y_old, hsq, pu_k):
    return pl.pallas_call(
        _update_body,
        grid=(_NP // _BN,),
        in_specs=[
            pl.BlockSpec((_NC, _BN, _DLAT), lambda i: (0, i, 0)),
            pl.BlockSpec((_BN, _DLAT), lambda i: (i, 0)),
            pl.BlockSpec((_BN, _DLAT), lambda i: (i, 0)),
            pl.BlockSpec((1, 1), lambda i: (0, 0)),
            pl.BlockSpec((_DLAT, _DIM), lambda i: (0, 0)),
        ],
        out_specs=[pl.BlockSpec((_BN, _DLAT), lambda i: (i, 0)),
                   pl.BlockSpec((_BN, _DIM), lambda i: (i, 0))],
        out_shape=[jax.ShapeDtypeStruct((_NP, _DLAT), jnp.float32),
                   jax.ShapeDtypeStruct((_NP, _DIM), jnp.float32)],
    )(accs, y, y_old, hsq, pu_k)


def kernel(x, batch, node_attr, edge_src, edge_dst, emb_table, fc1_W, fc1_b,
           dl_W, h, PU_K):
    del batch
    f32 = jnp.float32
    xp = jnp.pad(x.astype(f32), ((0, _NP - _N), (0, 0)))
    attr = jnp.pad(node_attr, (0, _NP - _N))
    srcf = jnp.pad(edge_src, (0, _EP - _E), constant_values=_NP - 1)
    dstf = jnp.pad(edge_dst, (0, _EP - _E), constant_values=_NP - 1)
    srcp = srcf.reshape(_NW, _NSUB, _CH)
    dstp = dstf.reshape(_NW, _NSUB, _CH)
    srcc = srcf.reshape(_CK, _NW, _NSUBC, _CH)
    dstc = dstf.reshape(_CK, _NW, _NSUBC, _CH)
    src2 = [jnp.concatenate([srcc[2 * i], srcc[2 * i + 1]], axis=1)
            for i in range(_CK // 2)]
    dst2 = [jnp.concatenate([dstc[2 * i], dstc[2 * i + 1]], axis=1)
            for i in range(_CK // 2)]
    zeros = jnp.zeros((_NC, _NP, _DLAT), f32)
    pu_kt = PU_K.T

    tps, tpd = _sc_typegather(attr, srcp, dstp)
    tps = tps.reshape(_EP, 1)
    tpd = tpd.reshape(_EP, 1)
    y = _tc_prep(xp, pu_kt)
    y_old = y
    xn = None
    for i in range(_L):
        ws = fc1_W[i][:, :_EMB].T
        wd = fc1_W[i][:, _EMB:2 * _EMB].T
        wcol = fc1_W[i][:, 2 * _EMB].reshape(1, _DLAT)
        bias = fc1_b[i].reshape(1, _DLAT)
        dlwt = dl_W[i].T
        part = zeros
        qps = []
        pqs = []
        for c in range(_CK):
            ys, yd = _sc_gather(y, srcc[c], dstc[c])
            base = jnp.full((1, 1), c * _EPC, jnp.int32)
            qp, pq = _tc_dense(ys, yd, tps[c * _EPC:(c + 1) * _EPC],
                               tpd[c * _EPC:(c + 1) * _EPC], emb_table,
                               ws, wd, wcol, bias, dlwt, PU_K, base)
            qps.append(qp)
            pqs.append(pq)
            if c % 2 == 1:
                part = _sc_scatter(qps[c - 1], qps[c], pqs[c - 1], pqs[c],
                                   dst2[c // 2], src2[c // 2], part)
        hsq = (h[i] * h[i]).reshape(1, 1)
        yn, xn = _tc_update(part, y, y_old, hsq, PU_K)
        y_old = y
        y = yn
    return xn[:_N]


# speedup vs baseline: 1.2271x; 1.0200x over previous
import dataclasses
import functools

import jax
import jax.numpy as jnp
from jax import lax
from jax.experimental import pallas as pl
from jax.experimental.pallas import tpu as pltpu
from jax.experimental.pallas import tpu_sc as plsc

_N = 10000
_E = 160000
_DLAT = 128
_DIM = 3
_L = 2
_EMB = 8
_NTYPES = 20
_MAXR = 50.0

_NC = 2
_NS = 16
_NW = _NC * _NS
_CH = 128
_NSUB = 40
_EPW = _CH * _NSUB
_EP = _EPW * _NW
_NP = 10240
_RPS = _NP // _NS
_LANES = 16
_CK = 4
_NSUBC = _NSUB // _CK
_EPC = _EP // _CK
_EPWC = _EPW // _CK
_GCH = 64
_GNSUBC = _EPWC // _GCH


def _sc_typegather(attr, idx_s, idx_d):
    mesh = plsc.VectorSubcoreMesh(core_axis_name="c", subcore_axis_name="s")
    out = [jax.ShapeDtypeStruct((_EP,), jnp.int32),
           jax.ShapeDtypeStruct((_EP,), jnp.int32)]
    cp = pltpu.CompilerParams()
    if "needs_layout_passes" in pltpu.CompilerParams.__dataclass_fields__:
        cp = dataclasses.replace(cp, needs_layout_passes=False)

    @functools.partial(
        pl.kernel, out_type=out, mesh=mesh, compiler_params=cp,
        scratch_types=[pltpu.VMEM((_NP,), jnp.int32),
                       pltpu.VMEM((_NSUB, _CH), jnp.int32),
                       pltpu.VMEM((_NSUB, _CH), jnp.int32),
                       pltpu.VMEM((_EPW,), jnp.int32),
                       pltpu.VMEM((_EPW,), jnp.int32)],
    )
    def k(a_hbm, is_hbm, id_hbm, os_hbm, od_hbm,
          a_v, iv_s, iv_d, ov_s, ov_d):
        wid = lax.axis_index("s") * _NC + lax.axis_index("c")
        base = wid * _EPW
        pltpu.sync_copy(a_hbm, a_v)
        pltpu.sync_copy(is_hbm.at[wid], iv_s)
        pltpu.sync_copy(id_hbm.at[wid], iv_d)

        @pl.loop(0, _NSUB)
        def _(j):
            @pl.loop(0, _CH, step=_LANES)
            def _(c):
                ts = plsc.load_gather(a_v, [iv_s[j, pl.ds(c, _LANES)]])
                td = plsc.load_gather(a_v, [iv_d[j, pl.ds(c, _LANES)]])
                ov_s[pl.ds(j * _CH + c, _LANES)] = ts
                ov_d[pl.ds(j * _CH + c, _LANES)] = td

        pltpu.sync_copy(ov_s, os_hbm.at[pl.ds(base, _EPW)])
        pltpu.sync_copy(ov_d, od_hbm.at[pl.ds(base, _EPW)])

    return k(attr, idx_s, idx_d)


def _sc_gather(tbl, idx_s, idx_d):
    mesh = plsc.VectorSubcoreMesh(core_axis_name="c", subcore_axis_name="s")
    out = [jax.ShapeDtypeStruct((_EPC, _DLAT), jnp.float32),
           jax.ShapeDtypeStruct((_EPC, _DLAT), jnp.float32)]

    @functools.partial(
        pl.kernel, out_type=out, mesh=mesh,
        scratch_types=[pltpu.VMEM((_GNSUBC, _GCH), jnp.int32),
                       pltpu.VMEM((_GNSUBC, _GCH), jnp.int32),
                       pltpu.VMEM((2, _GCH, _DLAT), jnp.float32),
                       pltpu.VMEM((2, _GCH, _DLAT), jnp.float32),
                       pltpu.VMEM_SHARED((_NP, _DLAT), jnp.float32),
                       pltpu.SemaphoreType.DMA,
                       pltpu.SemaphoreType.DMA],
    )
    def k(tbl_hbm, is_hbm, id_hbm, os_hbm, od_hbm,
          iv_s, iv_d, rows_s, rows_d, ytab, sem0, sem1):
        sid = lax.axis_index("s")
        wid = sid * _NC + lax.axis_index("c")
        base = wid * _EPWC
        rbase = sid * _RPS
        pltpu.sync_copy(tbl_hbm.at[pl.ds(rbase, _RPS)],
                        ytab.at[pl.ds(rbase, _RPS)])
        pltpu.sync_copy(is_hbm.at[wid], iv_s)
        pltpu.sync_copy(id_hbm.at[wid], iv_d)
        plsc.subcore_barrier()
        sems = (sem0, sem1)

        def issue(j, b):
            pltpu.make_async_copy(ytab.at[iv_s.at[j]], rows_s.at[b],
                                  sems[b]).start()
            pltpu.make_async_copy(ytab.at[iv_d.at[j]], rows_d.at[b],
                                  sems[b]).start()

        def drain(j, b):
            pltpu.make_async_copy(ytab.at[iv_s.at[j]], rows_s.at[b],
                                  sems[b]).wait()
            pltpu.make_async_copy(ytab.at[iv_d.at[j]], rows_d.at[b],
                                  sems[b]).wait()
            off = base + j * _GCH
            pltpu.sync_copy(rows_s.at[b], os_hbm.at[pl.ds(off, _GCH)])
            pltpu.sync_copy(rows_d.at[b], od_hbm.at[pl.ds(off, _GCH)])

        issue(0, 0)

        @pl.loop(0, _GNSUBC - 2, step=2)
        def _(j):
            issue(j + 1, 1)
            drain(j, 0)
            issue(j + 2, 0)
            drain(j + 1, 1)

        issue(_GNSUBC - 1, 1)
        drain(_GNSUBC - 2, 0)
        drain(_GNSUBC - 1, 1)

    return k(tbl, idx_s, idx_d)


def _sc_scatter(qpa, qpb, pqa, pqb, idx_d, idx_s, prev):
    mesh = plsc.VectorSubcoreMesh(core_axis_name="c", subcore_axis_name="s")
    out = jax.ShapeDtypeStruct((_NC, _NP, _DLAT), jnp.float32)

    @functools.partial(
        pl.kernel, out_type=out, mesh=mesh,
        scratch_types=[pltpu.VMEM((2 * _NSUBC, _CH), jnp.int32),
                       pltpu.VMEM((2 * _NSUBC, _CH), jnp.int32),
                       pltpu.VMEM((_CH, _DLAT), jnp.float32),
                       pltpu.VMEM((_CH, _DLAT), jnp.float32),
                       pltpu.VMEM_SHARED((_NP, _DLAT), jnp.float32)],
    )
    def k(qpa_hbm, qpb_hbm, pqa_hbm, pqb_hbm, id_hbm, is_hbm, z_hbm, o_hbm,
          iv_d, iv_s, rows_q, rows_p, acc):
        cid = lax.axis_index("c")
        sid = lax.axis_index("s")
        wid = sid * _NC + cid
        base = wid * _EPWC
        rbase = sid * _RPS
        pltpu.sync_copy(z_hbm.at[cid, pl.ds(rbase, _RPS)],
                        acc.at[pl.ds(rbase, _RPS)])
        pltpu.sync_copy(id_hbm.at[wid], iv_d)
        pltpu.sync_copy(is_hbm.at[wid], iv_s)
        plsc.subcore_barrier()

        for half, (qp_hbm, pq_hbm) in enumerate(((qpa_hbm, pqa_hbm),
                                                 (qpb_hbm, pqb_hbm))):
            @pl.loop(0, _NSUBC)
            def _(j, qp_hbm=qp_hbm, pq_hbm=pq_hbm, half=half):
                off = base + j * _CH
                pltpu.sync_copy(qp_hbm.at[pl.ds(off, _CH)], rows_q)
                pltpu.sync_copy(pq_hbm.at[pl.ds(off, _CH)], rows_p)
                pltpu.sync_copy(rows_q,
                                acc.at[iv_d.at[half * _NSUBC + j]], add=True)
                pltpu.sync_copy(rows_p,
                                acc.at[iv_s.at[half * _NSUBC + j]], add=True)

        plsc.subcore_barrier()
        pltpu.sync_copy(acc.at[pl.ds(rbase, _RPS)],
                        o_hbm.at[cid, pl.ds(rbase, _RPS)])

    return k(qpa, qpb, pqa, pqb, idx_d, idx_s, prev)


_BE = 1024


def _onehot(types):
    return (types == lax.broadcasted_iota(jnp.int32, (1, _NTYPES), 1)
            ).astype(jnp.float32)


def _dense_body(ys_ref, yd_ref, tps_ref, tpd_ref, e_ref, ws_ref, wd_ref,
                wc_ref, b_ref, w_ref, k_ref, base_ref, qp_ref, pq_ref):
    ys = ys_ref[...]
    yd = yd_ref[...]
    dif = ys - yd
    dv = jnp.dot(dif, k_ref[...], preferred_element_type=jnp.float32)
    r = jnp.sqrt(jnp.sum(dv * dv, axis=1, keepdims=True))
    u = 2.0 * (r / _MAXR - 1.0)
    c = (1.0 - jnp.cos(jnp.pi * u)) * 0.5
    c = jnp.where(u > 0.0, 0.0, c)
    c = jnp.where(u < -1.0, 1.0, c)
    w = c / r
    emb = e_ref[...]
    ps = jnp.dot(emb, ws_ref[...], preferred_element_type=jnp.float32)
    pd = jnp.dot(emb, wd_ref[...], preferred_element_type=jnp.float32)
    pre = (jnp.dot(_onehot(tps_ref[...]), ps,
                   preferred_element_type=jnp.float32)
           + jnp.dot(_onehot(tpd_ref[...]), pd,
                     preferred_element_type=jnp.float32)
           + w * wc_ref[...] + b_ref[...])
    wg = pre * jax.nn.sigmoid(pre)
    bf = jnp.bfloat16
    wgb = wg.astype(bf)
    gx = wgb * dif.astype(bf)
    ax = 0.5 * wgb * (ys + yd).astype(bf)
    dxe = jnp.concatenate([gx, ax, gx * ax, gx * gx, ax * ax], axis=1)
    wm = w_ref[...]
    t = jnp.dot(jnp.tanh(dxe), wm, preferred_element_type=jnp.float32)
    t = t - jnp.mean(t, axis=1, keepdims=True)
    t = t * lax.rsqrt(jnp.sum(t * t, axis=1, keepdims=True) + 1e-3)
    t = jnp.dot(jnp.tanh(t.astype(bf)), wm, preferred_element_type=jnp.float32)
    d2 = jnp.tanh(t.astype(bf))
    q = (wgb * d2[:, :_DLAT]).astype(jnp.float32)
    p = (0.5 * wgb * (d2[:, _DLAT:2 * _DLAT] + d2[:, 2 * _DLAT:3 * _DLAT]
                      + d2[:, 3 * _DLAT:4 * _DLAT] + d2[:, 4 * _DLAT:])
         ).astype(jnp.float32)
    eidx = (lax.broadcasted_iota(jnp.int32, (_BE, 1), 0)
            + pl.program_id(0) * _BE + base_ref[0, 0])
    mask = eidx < _E
    qp_ref[...] = jnp.where(mask, q + p, 0.0)
    pq_ref[...] = jnp.where(mask, p - q, 0.0)


def _tc_dense(ys, yd, tps, tpd, emb_table, ws, wd, wcol, bias, dlwt, pu_k,
              base):
    return pl.pallas_call(
        _dense_body,
        grid=(_EPC // _BE,),
        in_specs=[
            pl.BlockSpec((_BE, _DLAT), lambda i: (i, 0)),
            pl.BlockSpec((_BE, _DLAT), lambda i: (i, 0)),
            pl.BlockSpec((_BE, 1), lambda i: (i, 0)),
            pl.BlockSpec((_BE, 1), lambda i: (i, 0)),
            pl.BlockSpec((_NTYPES, _EMB), lambda i: (0, 0)),
            pl.BlockSpec((_EMB, _DLAT), lambda i: (0, 0)),
            pl.BlockSpec((_EMB, _DLAT), lambda i: (0, 0)),
            pl.BlockSpec((1, _DLAT), lambda i: (0, 0)),
            pl.BlockSpec((1, _DLAT), lambda i: (0, 0)),
            pl.BlockSpec((5 * _DLAT, 5 * _DLAT), lambda i: (0, 0)),
            pl.BlockSpec((_DLAT, _DIM), lambda i: (0, 0)),
            pl.BlockSpec((1, 1), lambda i: (0, 0)),
        ],
        out_specs=[pl.BlockSpec((_BE, _DLAT), lambda i: (i, 0)),
                   pl.BlockSpec((_BE, _DLAT), lambda i: (i, 0))],
        out_shape=[jax.ShapeDtypeStruct((_EPC, _DLAT), jnp.float32),
                   jax.ShapeDtypeStruct((_EPC, _DLAT), jnp.float32)],
    )(ys, yd, tps, tpd, emb_table, ws, wd, wcol, bias,
      dlwt.astype(jnp.bfloat16), pu_k, base)


_BN = 1024


def _prep_body(x_ref, k_ref, y_ref):
    y_ref[...] = jnp.dot(x_ref[...], k_ref[...],
                         preferred_element_type=jnp.float32)


def _tc_prep(xp, pu_kt):
    return pl.pallas_call(
        _prep_body,
        grid=(_NP // _BN,),
        in_specs=[
            pl.BlockSpec((_BN, _DIM), lambda i: (i, 0)),
            pl.BlockSpec((_DIM, _DLAT), lambda i: (0, 0)),
        ],
        out_specs=pl.BlockSpec((_BN, _DLAT), lambda i: (i, 0)),
        out_shape=jax.ShapeDtypeStruct((_NP, _DLAT), jnp.float32),
    )(xp, pu_kt)


def _update_body(acc_ref, y_ref, yo_ref, h_ref, k_ref, yn_ref, x_ref):
    accsum = acc_ref[0] + acc_ref[1]
    yn = 2.0 * y_ref[...] - yo_ref[...] - h_ref[0, 0] * accsum
    yn_ref[...] = yn
    x_ref[...] = jnp.dot(yn, k_ref[...], preferred_element_type=jnp.float32)


def _tc_update(accs, y, y_old, hsq, pu_k):
    return pl.pallas_call(
        _update_body,
        grid=(_NP // _BN,),
        in_specs=[
            pl.BlockSpec((_NC, _BN, _DLAT), lambda i: (0, i, 0)),
            pl.BlockSpec((_BN, _DLAT), lambda i: (i, 0)),
            pl.BlockSpec((_BN, _DLAT), lambda i: (i, 0)),
            pl.BlockSpec((1, 1), lambda i: (0, 0)),
            pl.BlockSpec((_DLAT, _DIM), lambda i: (0, 0)),
        ],
        out_specs=[pl.BlockSpec((_BN, _DLAT), lambda i: (i, 0)),
                   pl.BlockSpec((_BN, _DIM), lambda i: (i, 0))],
        out_shape=[jax.ShapeDtypeStruct((_NP, _DLAT), jnp.float32),
                   jax.ShapeDtypeStruct((_NP, _DIM), jnp.float32)],
    )(accs, y, y_old, hsq, pu_k)


def kernel(x, batch, node_attr, edge_src, edge_dst, emb_table, fc1_W, fc1_b,
           dl_W, h, PU_K):
    del batch
    f32 = jnp.float32
    xp = jnp.pad(x.astype(f32), ((0, _NP - _N), (0, 0)))
    attr = jnp.pad(node_attr, (0, _NP - _N))
    srcf = jnp.pad(edge_src, (0, _EP - _E), constant_values=_NP - 1)
    dstf = jnp.pad(edge_dst, (0, _EP - _E), constant_values=_NP - 1)
    srcp = srcf.reshape(_NW, _NSUB, _CH)
    dstp = dstf.reshape(_NW, _NSUB, _CH)
    srcc = srcf.reshape(_CK, _NW, _NSUBC, _CH)
    dstc = dstf.reshape(_CK, _NW, _NSUBC, _CH)
    srcg = srcf.reshape(_CK, _NW, _GNSUBC, _GCH)
    dstg = dstf.reshape(_CK, _NW, _GNSUBC, _GCH)
    src2 = [jnp.concatenate([srcc[2 * i], srcc[2 * i + 1]], axis=1)
            for i in range(_CK // 2)]
    dst2 = [jnp.concatenate([dstc[2 * i], dstc[2 * i + 1]], axis=1)
            for i in range(_CK // 2)]
    zeros = jnp.zeros((_NC, _NP, _DLAT), f32)
    pu_kt = PU_K.T

    tps, tpd = _sc_typegather(attr, srcp, dstp)
    tps = tps.reshape(_EP, 1)
    tpd = tpd.reshape(_EP, 1)
    y = _tc_prep(xp, pu_kt)
    y_old = y
    xn = None
    for i in range(_L):
        ws = fc1_W[i][:, :_EMB].T
        wd = fc1_W[i][:, _EMB:2 * _EMB].T
        wcol = fc1_W[i][:, 2 * _EMB].reshape(1, _DLAT)
        bias = fc1_b[i].reshape(1, _DLAT)
        dlwt = dl_W[i].T
        part = zeros
        qps = []
        pqs = []
        for c in range(_CK):
            ys, yd = _sc_gather(y, srcg[c], dstg[c])
            base = jnp.full((1, 1), c * _EPC, jnp.int32)
            qp, pq = _tc_dense(ys, yd, tps[c * _EPC:(c + 1) * _EPC],
                               tpd[c * _EPC:(c + 1) * _EPC], emb_table,
                               ws, wd, wcol, bias, dlwt, PU_K, base)
            qps.append(qp)
            pqs.append(pq)
            if c % 2 == 1:
                part = _sc_scatter(qps[c - 1], qps[c], pqs[c - 1], pqs[c],
                                   dst2[c // 2], src2[c // 2], part)
        hsq = (h[i] * h[i]).reshape(1, 1)
        yn, xn = _tc_update(part, y, y_old, hsq, PU_K)
        y_old = y
        y = yn
    return xn[:_N]


# speedup vs baseline: 1.2488x; 1.0177x over previous
import dataclasses
import functools

import jax
import jax.numpy as jnp
from jax import lax
from jax.experimental import pallas as pl
from jax.experimental.pallas import tpu as pltpu
from jax.experimental.pallas import tpu_sc as plsc

_N = 10000
_E = 160000
_DLAT = 128
_DIM = 3
_L = 2
_EMB = 8
_NTYPES = 20
_MAXR = 50.0

_NC = 2
_NS = 16
_NW = _NC * _NS
_CH = 128
_NSUB = 40
_EPW = _CH * _NSUB
_EP = _EPW * _NW
_NP = 10240
_RPS = _NP // _NS
_LANES = 16
_CK = 4
_NSUBC = _NSUB // _CK
_EPC = _EP // _CK
_EPWC = _EPW // _CK
_GCH = 64
_GNSUBC = _EPWC // _GCH


def _sc_typegather(attr, idx_s, idx_d):
    mesh = plsc.VectorSubcoreMesh(core_axis_name="c", subcore_axis_name="s")
    out = [jax.ShapeDtypeStruct((_EP,), jnp.int32),
           jax.ShapeDtypeStruct((_EP,), jnp.int32)]
    cp = pltpu.CompilerParams()
    if "needs_layout_passes" in pltpu.CompilerParams.__dataclass_fields__:
        cp = dataclasses.replace(cp, needs_layout_passes=False)

    @functools.partial(
        pl.kernel, out_type=out, mesh=mesh, compiler_params=cp,
        scratch_types=[pltpu.VMEM((_NP,), jnp.int32),
                       pltpu.VMEM((_NSUB, _CH), jnp.int32),
                       pltpu.VMEM((_NSUB, _CH), jnp.int32),
                       pltpu.VMEM((_EPW,), jnp.int32),
                       pltpu.VMEM((_EPW,), jnp.int32)],
    )
    def k(a_hbm, is_hbm, id_hbm, os_hbm, od_hbm,
          a_v, iv_s, iv_d, ov_s, ov_d):
        wid = lax.axis_index("s") * _NC + lax.axis_index("c")
        base = wid * _EPW
        pltpu.sync_copy(a_hbm, a_v)
        pltpu.sync_copy(is_hbm.at[wid], iv_s)
        pltpu.sync_copy(id_hbm.at[wid], iv_d)

        @pl.loop(0, _NSUB)
        def _(j):
            @pl.loop(0, _CH, step=_LANES)
            def _(c):
                ts = plsc.load_gather(a_v, [iv_s[j, pl.ds(c, _LANES)]])
                td = plsc.load_gather(a_v, [iv_d[j, pl.ds(c, _LANES)]])
                ov_s[pl.ds(j * _CH + c, _LANES)] = ts
                ov_d[pl.ds(j * _CH + c, _LANES)] = td

        pltpu.sync_copy(ov_s, os_hbm.at[pl.ds(base, _EPW)])
        pltpu.sync_copy(ov_d, od_hbm.at[pl.ds(base, _EPW)])

    return k(attr, idx_s, idx_d)


def _sc_gather(tbl, idx_s, idx_d):
    mesh = plsc.VectorSubcoreMesh(core_axis_name="c", subcore_axis_name="s")
    out = [jax.ShapeDtypeStruct((_EPC, _DLAT), jnp.float32),
           jax.ShapeDtypeStruct((_EPC, _DLAT), jnp.float32)]

    @functools.partial(
        pl.kernel, out_type=out, mesh=mesh,
        scratch_types=[pltpu.VMEM((_GNSUBC, _GCH), jnp.int32),
                       pltpu.VMEM((_GNSUBC, _GCH), jnp.int32),
                       pltpu.VMEM((2, _GCH, _DLAT), jnp.float32),
                       pltpu.VMEM((2, _GCH, _DLAT), jnp.float32),
                       pltpu.VMEM_SHARED((_NP, _DLAT), jnp.float32),
                       pltpu.SemaphoreType.DMA,
                       pltpu.SemaphoreType.DMA],
    )
    def k(tbl_hbm, is_hbm, id_hbm, os_hbm, od_hbm,
          iv_s, iv_d, rows_s, rows_d, ytab, sem0, sem1):
        sid = lax.axis_index("s")
        wid = sid * _NC + lax.axis_index("c")
        base = wid * _EPWC
        rbase = sid * _RPS
        pltpu.sync_copy(tbl_hbm.at[pl.ds(rbase, _RPS)],
                        ytab.at[pl.ds(rbase, _RPS)])
        pltpu.sync_copy(is_hbm.at[wid], iv_s)
        pltpu.sync_copy(id_hbm.at[wid], iv_d)
        plsc.subcore_barrier()
        sems = (sem0, sem1)

        def issue(j, b):
            pltpu.make_async_copy(ytab.at[iv_s.at[j]], rows_s.at[b],
                                  sems[b]).start()
            pltpu.make_async_copy(ytab.at[iv_d.at[j]], rows_d.at[b],
                                  sems[b]).start()

        def drain(j, b):
            pltpu.make_async_copy(ytab.at[iv_s.at[j]], rows_s.at[b],
                                  sems[b]).wait()
            pltpu.make_async_copy(ytab.at[iv_d.at[j]], rows_d.at[b],
                                  sems[b]).wait()
            off = base + j * _GCH
            pltpu.sync_copy(rows_s.at[b], os_hbm.at[pl.ds(off, _GCH)])
            pltpu.sync_copy(rows_d.at[b], od_hbm.at[pl.ds(off, _GCH)])

        issue(0, 0)

        @pl.loop(0, _GNSUBC - 2, step=2)
        def _(j):
            issue(j + 1, 1)
            drain(j, 0)
            issue(j + 2, 0)
            drain(j + 1, 1)

        issue(_GNSUBC - 1, 1)
        drain(_GNSUBC - 2, 0)
        drain(_GNSUBC - 1, 1)

    return k(tbl, idx_s, idx_d)


def _sc_scatter(qpa, qpb, pqa, pqb, idx_d, idx_s, prev):
    mesh = plsc.VectorSubcoreMesh(core_axis_name="c", subcore_axis_name="s")
    out = jax.ShapeDtypeStruct((_NC, _NP, _DLAT), jnp.float32)

    nsw = _EPWC // _GCH

    @functools.partial(
        pl.kernel, out_type=out, mesh=mesh,
        scratch_types=[pltpu.VMEM((2 * nsw, _GCH), jnp.int32),
                       pltpu.VMEM((2 * nsw, _GCH), jnp.int32),
                       pltpu.VMEM((2, _GCH, _DLAT), jnp.float32),
                       pltpu.VMEM((2, _GCH, _DLAT), jnp.float32),
                       pltpu.VMEM_SHARED((_NP, _DLAT), jnp.float32),
                       pltpu.SemaphoreType.DMA,
                       pltpu.SemaphoreType.DMA],
    )
    def k(qpa_hbm, qpb_hbm, pqa_hbm, pqb_hbm, id_hbm, is_hbm, z_hbm, o_hbm,
          iv_d, iv_s, rows_q, rows_p, acc, sem0, sem1):
        cid = lax.axis_index("c")
        sid = lax.axis_index("s")
        wid = sid * _NC + cid
        base = wid * _EPWC
        rbase = sid * _RPS
        pltpu.sync_copy(z_hbm.at[cid, pl.ds(rbase, _RPS)],
                        acc.at[pl.ds(rbase, _RPS)])
        pltpu.sync_copy(id_hbm.at[wid], iv_d)
        pltpu.sync_copy(is_hbm.at[wid], iv_s)
        plsc.subcore_barrier()
        sems = (sem0, sem1)

        for half, (qp_hbm, pq_hbm) in enumerate(((qpa_hbm, pqa_hbm),
                                                 (qpb_hbm, pqb_hbm))):
            def load(j, b, qp_hbm=qp_hbm, pq_hbm=pq_hbm):
                off = base + j * _GCH
                pltpu.make_async_copy(qp_hbm.at[pl.ds(off, _GCH)],
                                      rows_q.at[b], sems[b]).start()
                pltpu.make_async_copy(pq_hbm.at[pl.ds(off, _GCH)],
                                      rows_p.at[b], sems[b]).start()

            def scat(j, b, qp_hbm=qp_hbm, pq_hbm=pq_hbm, half=half):
                off = base + j * _GCH
                pltpu.make_async_copy(qp_hbm.at[pl.ds(off, _GCH)],
                                      rows_q.at[b], sems[b]).wait()
                pltpu.make_async_copy(pq_hbm.at[pl.ds(off, _GCH)],
                                      rows_p.at[b], sems[b]).wait()
                row = half * nsw + j
                pltpu.sync_copy(rows_q.at[b], acc.at[iv_d.at[row]], add=True)
                pltpu.sync_copy(rows_p.at[b], acc.at[iv_s.at[row]], add=True)

            load(0, 0)

            @pl.loop(0, nsw - 2, step=2)
            def _(j, load=load, scat=scat):
                load(j + 1, 1)
                scat(j, 0)
                load(j + 2, 0)
                scat(j + 1, 1)

            load(nsw - 1, 1)
            scat(nsw - 2, 0)
            scat(nsw - 1, 1)

        plsc.subcore_barrier()
        pltpu.sync_copy(acc.at[pl.ds(rbase, _RPS)],
                        o_hbm.at[cid, pl.ds(rbase, _RPS)])

    return k(qpa, qpb, pqa, pqb, idx_d, idx_s, prev)


_BE = 1024


def _onehot(types):
    return (types == lax.broadcasted_iota(jnp.int32, (1, _NTYPES), 1)
            ).astype(jnp.float32)


def _dense_body(ys_ref, yd_ref, tps_ref, tpd_ref, e_ref, ws_ref, wd_ref,
                wc_ref, b_ref, w_ref, k_ref, base_ref, qp_ref, pq_ref):
    ys = ys_ref[...]
    yd = yd_ref[...]
    dif = ys - yd
    dv = jnp.dot(dif, k_ref[...], preferred_element_type=jnp.float32)
    r = jnp.sqrt(jnp.sum(dv * dv, axis=1, keepdims=True))
    u = 2.0 * (r / _MAXR - 1.0)
    c = (1.0 - jnp.cos(jnp.pi * u)) * 0.5
    c = jnp.where(u > 0.0, 0.0, c)
    c = jnp.where(u < -1.0, 1.0, c)
    w = c / r
    emb = e_ref[...]
    ps = jnp.dot(emb, ws_ref[...], preferred_element_type=jnp.float32)
    pd = jnp.dot(emb, wd_ref[...], preferred_element_type=jnp.float32)
    pre = (jnp.dot(_onehot(tps_ref[...]), ps,
                   preferred_element_type=jnp.float32)
           + jnp.dot(_onehot(tpd_ref[...]), pd,
                     preferred_element_type=jnp.float32)
           + w * wc_ref[...] + b_ref[...])
    wg = pre * jax.nn.sigmoid(pre)
    bf = jnp.bfloat16
    wgb = wg.astype(bf)
    gx = wgb * dif.astype(bf)
    ax = 0.5 * wgb * (ys + yd).astype(bf)
    dxe = jnp.concatenate([gx, ax, gx * ax, gx * gx, ax * ax], axis=1)
    wm = w_ref[...]
    t = jnp.dot(jnp.tanh(dxe), wm, preferred_element_type=jnp.float32)
    t = t - jnp.mean(t, axis=1, keepdims=True)
    t = t * lax.rsqrt(jnp.sum(t * t, axis=1, keepdims=True) + 1e-3)
    t = jnp.dot(jnp.tanh(t.astype(bf)), wm, preferred_element_type=jnp.float32)
    d2 = jnp.tanh(t.astype(bf))
    q = (wgb * d2[:, :_DLAT]).astype(jnp.float32)
    p = (0.5 * wgb * (d2[:, _DLAT:2 * _DLAT] + d2[:, 2 * _DLAT:3 * _DLAT]
                      + d2[:, 3 * _DLAT:4 * _DLAT] + d2[:, 4 * _DLAT:])
         ).astype(jnp.float32)
    eidx = (lax.broadcasted_iota(jnp.int32, (_BE, 1), 0)
            + pl.program_id(0) * _BE + base_ref[0, 0])
    mask = eidx < _E
    qp_ref[...] = jnp.where(mask, q + p, 0.0)
    pq_ref[...] = jnp.where(mask, p - q, 0.0)


def _tc_dense(ys, yd, tps, tpd, emb_table, ws, wd, wcol, bias, dlwt, pu_k,
              base):
    return pl.pallas_call(
        _dense_body,
        grid=(_EPC // _BE,),
        in_specs=[
            pl.BlockSpec((_BE, _DLAT), lambda i: (i, 0)),
            pl.BlockSpec((_BE, _DLAT), lambda i: (i, 0)),
            pl.BlockSpec((_BE, 1), lambda i: (i, 0)),
            pl.BlockSpec((_BE, 1), lambda i: (i, 0)),
            pl.BlockSpec((_NTYPES, _EMB), lambda i: (0, 0)),
            pl.BlockSpec((_EMB, _DLAT), lambda i: (0, 0)),
            pl.BlockSpec((_EMB, _DLAT), lambda i: (0, 0)),
            pl.BlockSpec((1, _DLAT), lambda i: (0, 0)),
            pl.BlockSpec((1, _DLAT), lambda i: (0, 0)),
            pl.BlockSpec((5 * _DLAT, 5 * _DLAT), lambda i: (0, 0)),
            pl.BlockSpec((_DLAT, _DIM), lambda i: (0, 0)),
            pl.BlockSpec((1, 1), lambda i: (0, 0)),
        ],
        out_specs=[pl.BlockSpec((_BE, _DLAT), lambda i: (i, 0)),
                   pl.BlockSpec((_BE, _DLAT), lambda i: (i, 0))],
        out_shape=[jax.ShapeDtypeStruct((_EPC, _DLAT), jnp.float32),
                   jax.ShapeDtypeStruct((_EPC, _DLAT), jnp.float32)],
    )(ys, yd, tps, tpd, emb_table, ws, wd, wcol, bias,
      dlwt.astype(jnp.bfloat16), pu_k, base)


_BN = 1024


def _prep_body(x_ref, k_ref, y_ref):
    y_ref[...] = jnp.dot(x_ref[...], k_ref[...],
                         preferred_element_type=jnp.float32)


def _tc_prep(xp, pu_kt):
    return pl.pallas_call(
        _prep_body,
        grid=(_NP // _BN,),
        in_specs=[
            pl.BlockSpec((_BN, _DIM), lambda i: (i, 0)),
            pl.BlockSpec((_DIM, _DLAT), lambda i: (0, 0)),
        ],
        out_specs=pl.BlockSpec((_BN, _DLAT), lambda i: (i, 0)),
        out_shape=jax.ShapeDtypeStruct((_NP, _DLAT), jnp.float32),
    )(xp, pu_kt)


def _update_body(acc_ref, y_ref, yo_ref, h_ref, k_ref, yn_ref, x_ref):
    accsum = acc_ref[0] + acc_ref[1]
    yn = 2.0 * y_ref[...] - yo_ref[...] - h_ref[0, 0] * accsum
    yn_ref[...] = yn
    x_ref[...] = jnp.dot(yn, k_ref[...], preferred_element_type=jnp.float32)


def _tc_update(accs, y, y_old, hsq, pu_k):
    return pl.pallas_call(
        _update_body,
        grid=(_NP // _BN,),
        in_specs=[
            pl.BlockSpec((_NC, _BN, _DLAT), lambda i: (0, i, 0)),
            pl.BlockSpec((_BN, _DLAT), lambda i: (i, 0)),
            pl.BlockSpec((_BN, _DLAT), lambda i: (i, 0)),
            pl.BlockSpec((1, 1), lambda i: (0, 0)),
            pl.BlockSpec((_DLAT, _DIM), lambda i: (0, 0)),
        ],
        out_specs=[pl.BlockSpec((_BN, _DLAT), lambda i: (i, 0)),
                   pl.BlockSpec((_BN, _DIM), lambda i: (i, 0))],
        out_shape=[jax.ShapeDtypeStruct((_NP, _DLAT), jnp.float32),
                   jax.ShapeDtypeStruct((_NP, _DIM), jnp.float32)],
    )(accs, y, y_old, hsq, pu_k)


def kernel(x, batch, node_attr, edge_src, edge_dst, emb_table, fc1_W, fc1_b,
           dl_W, h, PU_K):
    del batch
    f32 = jnp.float32
    xp = jnp.pad(x.astype(f32), ((0, _NP - _N), (0, 0)))
    attr = jnp.pad(node_attr, (0, _NP - _N))
    srcf = jnp.pad(edge_src, (0, _EP - _E), constant_values=_NP - 1)
    dstf = jnp.pad(edge_dst, (0, _EP - _E), constant_values=_NP - 1)
    srcp = srcf.reshape(_NW, _NSUB, _CH)
    dstp = dstf.reshape(_NW, _NSUB, _CH)
    srcc = srcf.reshape(_CK, _NW, _NSUBC, _CH)
    dstc = dstf.reshape(_CK, _NW, _NSUBC, _CH)
    srcg = srcf.reshape(_CK, _NW, _GNSUBC, _GCH)
    dstg = dstf.reshape(_CK, _NW, _GNSUBC, _GCH)
    nsw2 = 2 * (_EPWC // _GCH)
    src2 = [jnp.concatenate([srcc[2 * i], srcc[2 * i + 1]], axis=1)
            .reshape(_NW, nsw2, _GCH) for i in range(_CK // 2)]
    dst2 = [jnp.concatenate([dstc[2 * i], dstc[2 * i + 1]], axis=1)
            .reshape(_NW, nsw2, _GCH) for i in range(_CK // 2)]
    zeros = jnp.zeros((_NC, _NP, _DLAT), f32)
    pu_kt = PU_K.T

    tps, tpd = _sc_typegather(attr, srcp, dstp)
    tps = tps.reshape(_EP, 1)
    tpd = tpd.reshape(_EP, 1)
    y = _tc_prep(xp, pu_kt)
    y_old = y
    xn = None
    for i in range(_L):
        ws = fc1_W[i][:, :_EMB].T
        wd = fc1_W[i][:, _EMB:2 * _EMB].T
        wcol = fc1_W[i][:, 2 * _EMB].reshape(1, _DLAT)
        bias = fc1_b[i].reshape(1, _DLAT)
        dlwt = dl_W[i].T
        part = zeros
        qps = []
        pqs = []
        for c in range(_CK):
            ys, yd = _sc_gather(y, srcg[c], dstg[c])
            base = jnp.full((1, 1), c * _EPC, jnp.int32)
            qp, pq = _tc_dense(ys, yd, tps[c * _EPC:(c + 1) * _EPC],
                               tpd[c * _EPC:(c + 1) * _EPC], emb_table,
                               ws, wd, wcol, bias, dlwt, PU_K, base)
            qps.append(qp)
            pqs.append(pq)
            if c % 2 == 1:
                part = _sc_scatter(qps[c - 1], qps[c], pqs[c - 1], pqs[c],
                                   dst2[c // 2], src2[c // 2], part)
        hsq = (h[i] * h[i]).reshape(1, 1)
        yn, xn = _tc_update(part, y, y_old, hsq, PU_K)
        y_old = y
        y = yn
    return xn[:_N]
